# initial kernel scaffold (unmeasured)
import jax
import jax.numpy as jnp
from jax import lax
from jax.experimental import pallas as pl
from jax.experimental.pallas import tpu as pltpu

N_DEV = 4
M = 4096
D = 1024
CH = M // N_DEV


def kernel(x, W1, W2):
    m, k = x.shape
    d = W1.shape[1]
    n = W2.shape[1]

    def body(x_ref, w1_ref, w2_ref, out_ref, comm_ref, tmp_ref,
             rs_send_sems, rs_recv_sems, ag_send_sems, ag_recv_sems):
        my = lax.axis_index("i")
        left = (my - 1) % N_DEV
        right = (my + 1) % N_DEV

        barrier_sem = pltpu.get_barrier_semaphore()
        for nbr in [left, right]:
            pl.semaphore_signal(
                barrier_sem, inc=1,
                device_id=(nbr,), device_id_type=pl.DeviceIdType.MESH,
            )
        pl.semaphore_wait(barrier_sem, 2)

        def gemm1_chunk(c):
            return jnp.dot(
                x_ref[pl.ds(c * CH, CH), :], w1_ref[:, :],
                preferred_element_type=jnp.float32,
            )

        tmp_ref[:, :] = gemm1_chunk(my)
        for s in range(3):
            src = tmp_ref if s == 0 else comm_ref.at[s - 1]
            rdma = pltpu.make_async_remote_copy(
                src_ref=src,
                dst_ref=comm_ref.at[s],
                send_sem=rs_send_sems.at[s],
                recv_sem=rs_recv_sems.at[s],
                device_id=(right,),
                device_id_type=pl.DeviceIdType.MESH,
            )
            rdma.start()
            rdma.wait()
            c = (my - s - 1) % N_DEV
            comm_ref[s, :, :] = comm_ref[s, :, :] + gemm1_chunk(c)

        own = (my + 1) % N_DEV
        out_ref[pl.ds(own * CH, CH), :] = jnp.dot(
            comm_ref[2, :, :], w2_ref[:, :], preferred_element_type=jnp.float32,
        )

        ag_slot = [3, 0, 1]
        for s in range(3):
            src = comm_ref.at[2] if s == 0 else comm_ref.at[ag_slot[s - 1]]
            rdma = pltpu.make_async_remote_copy(
                src_ref=src,
                dst_ref=comm_ref.at[ag_slot[s]],
                send_sem=ag_send_sems.at[s],
                recv_sem=ag_recv_sems.at[s],
                device_id=(right,),
                device_id_type=pl.DeviceIdType.MESH,
            )
            rdma.start()
            rdma.wait()
            c = (my - s) % N_DEV
            out_ref[pl.ds(c * CH, CH), :] = jnp.dot(
                comm_ref[ag_slot[s], :, :], w2_ref[:, :],
                preferred_element_type=jnp.float32,
            )

    return pl.pallas_call(
        body,
        out_shape=jax.ShapeDtypeStruct((m, n), jnp.float32),
        in_specs=[
            pl.BlockSpec(memory_space=pltpu.VMEM),
            pl.BlockSpec(memory_space=pltpu.VMEM),
            pl.BlockSpec(memory_space=pltpu.VMEM),
        ],
        out_specs=pl.BlockSpec(memory_space=pltpu.VMEM),
        scratch_shapes=[
            pltpu.VMEM((4, CH, d), jnp.float32),
            pltpu.VMEM((CH, d), jnp.float32),
            pltpu.SemaphoreType.DMA((3,)),
            pltpu.SemaphoreType.DMA((3,)),
            pltpu.SemaphoreType.DMA((3,)),
            pltpu.SemaphoreType.DMA((3,)),
        ],
        compiler_params=pltpu.CompilerParams(collective_id=0),
    )(x, W1, W2)


# baseline (device time: 330269 ns/iter reference)
import jax
import jax.numpy as jnp
from jax import lax
from jax.experimental import pallas as pl
from jax.experimental.pallas import tpu as pltpu

N_DEV = 4
M = 4096
D = 1024
CH = M // N_DEV


def kernel(x, W1, W2):
    m, k = x.shape
    d = W1.shape[1]
    n = W2.shape[1]

    def body(x_ref, w1_ref, w2_ref, out_ref, comm_ref, tmp_ref,
             rs_send_sems, rs_recv_sems, ag_send_sems, ag_recv_sems):
        my = lax.axis_index("i")
        left = (my - 1) % N_DEV
        right = (my + 1) % N_DEV

        barrier_sem = pltpu.get_barrier_semaphore()
        for nbr in [left, right]:
            pl.semaphore_signal(
                barrier_sem, inc=1,
                device_id=(nbr,), device_id_type=pl.DeviceIdType.MESH,
            )
        pl.semaphore_wait(barrier_sem, 2)

        def gemm1_chunk(c):
            return jnp.dot(
                x_ref[pl.ds(c * CH, CH), :], w1_ref[:, :],
                preferred_element_type=jnp.float32,
            )

        tmp_ref[:, :] = gemm1_chunk(my)
        for s in range(3):
            src = tmp_ref if s == 0 else comm_ref.at[s - 1]
            rdma = pltpu.make_async_remote_copy(
                src_ref=src,
                dst_ref=comm_ref.at[s],
                send_sem=rs_send_sems.at[s],
                recv_sem=rs_recv_sems.at[s],
                device_id=(right,),
                device_id_type=pl.DeviceIdType.MESH,
            )
            rdma.start()
            rdma.wait()
            c = (my - s - 1) % N_DEV
            comm_ref[s, :, :] = comm_ref[s, :, :] + gemm1_chunk(c)

        own = (my + 1) % N_DEV
        out_ref[pl.ds(own * CH, CH), :] = jnp.dot(
            comm_ref[2, :, :], w2_ref[:, :], preferred_element_type=jnp.float32,
        )

        ag_slot = [3, 0, 1]
        for s in range(3):
            src = comm_ref.at[2] if s == 0 else comm_ref.at[ag_slot[s - 1]]
            rdma = pltpu.make_async_remote_copy(
                src_ref=src,
                dst_ref=comm_ref.at[ag_slot[s]],
                send_sem=ag_send_sems.at[s],
                recv_sem=ag_recv_sems.at[s],
                device_id=(right,),
                device_id_type=pl.DeviceIdType.MESH,
            )
            rdma.start()
            rdma.wait()
            c = (my - s) % N_DEV
            out_ref[pl.ds(c * CH, CH), :] = jnp.dot(
                comm_ref[ag_slot[s], :, :], w2_ref[:, :],
                preferred_element_type=jnp.float32,
            )

    return pl.pallas_call(
        body,
        out_shape=jax.ShapeDtypeStruct((m, n), jnp.float32),
        in_specs=[
            pl.BlockSpec(memory_space=pltpu.VMEM),
            pl.BlockSpec(memory_space=pltpu.VMEM),
            pl.BlockSpec(memory_space=pltpu.VMEM),
        ],
        out_specs=pl.BlockSpec(memory_space=pltpu.VMEM),
        scratch_shapes=[
            pltpu.VMEM((4, CH, d), jnp.float32),
            pltpu.VMEM((CH, d), jnp.float32),
            pltpu.SemaphoreType.DMA((3,)),
            pltpu.SemaphoreType.DMA((3,)),
            pltpu.SemaphoreType.DMA((3,)),
            pltpu.SemaphoreType.DMA((3,)),
        ],
        compiler_params=pltpu.CompilerParams(
            collective_id=0,
            vmem_limit_bytes=64 * 1024 * 1024,
        ),
    )(x, W1, W2)


# device time: 183043 ns/iter; 1.8043x vs baseline; 1.8043x over previous
import jax
import jax.numpy as jnp
from jax import lax
from jax.experimental import pallas as pl
from jax.experimental.pallas import tpu as pltpu

N_DEV = 4
CH = 1024
HF = CH // 2


def kernel(x, W1, W2):
    m, k = x.shape
    d = W1.shape[1]
    n = W2.shape[1]

    def body(x_ref, w1_ref, w2_ref, out_ref, cw_ref, ccw_ref,
             tA_ref, tB_ref,
             cw_send_sems, cw_recv_sems, ccw_send_sems, ccw_recv_sems,
             agcw_send_sems, agcw_recv_sems, agccw_send_sems,
             agccw_recv_sems):
        my = lax.axis_index("i")
        left = (my - 1) % N_DEV
        right = (my + 1) % N_DEV

        barrier_sem = pltpu.get_barrier_semaphore()
        for nbr in [left, right]:
            pl.semaphore_signal(
                barrier_sem, inc=1,
                device_id=(nbr,), device_id_type=pl.DeviceIdType.MESH,
            )
        pl.semaphore_wait(barrier_sem, 2)

        def gemm1_top(c):
            return jnp.dot(
                x_ref[pl.ds(c * CH, HF), :], w1_ref[:, :],
                preferred_element_type=jnp.float32,
            )

        def gemm1_bot(c):
            return jnp.dot(
                x_ref[pl.ds(c * CH + HF, HF), :], w1_ref[:, :],
                preferred_element_type=jnp.float32,
            )

        def gemm2_top(c, buf):
            out_ref[pl.ds(c * CH, HF), :] = jnp.dot(
                buf, w2_ref[:, :], preferred_element_type=jnp.float32,
            )

        def gemm2_bot(c, buf):
            out_ref[pl.ds(c * CH + HF, HF), :] = jnp.dot(
                buf, w2_ref[:, :], preferred_element_type=jnp.float32,
            )

        def rdma(src, dst, ssem, rsem, target):
            return pltpu.make_async_remote_copy(
                src_ref=src, dst_ref=dst, send_sem=ssem, recv_sem=rsem,
                device_id=(target,), device_id_type=pl.DeviceIdType.MESH,
            )

        tA_ref[:, :] = gemm1_top(my)
        tB_ref[:, :] = gemm1_bot(my)
        for s in range(3):
            r_cw = rdma(tA_ref if s == 0 else cw_ref.at[s - 1],
                        cw_ref.at[s], cw_send_sems.at[s], cw_recv_sems.at[s],
                        right)
            r_ccw = rdma(tB_ref if s == 0 else ccw_ref.at[s - 1],
                         ccw_ref.at[s], ccw_send_sems.at[s],
                         ccw_recv_sems.at[s], left)
            r_cw.start()
            r_ccw.start()
            if s > 0:
                tA_ref[:, :] = gemm1_top((my - s - 1) % N_DEV)
                tB_ref[:, :] = gemm1_bot((my + s + 1) % N_DEV)
            r_cw.wait()
            r_ccw.wait()
            if s == 0:
                cw_ref[0, :, :] = cw_ref[0, :, :] + gemm1_top((my - 1) % N_DEV)
                ccw_ref[0, :, :] = ccw_ref[0, :, :] + gemm1_bot((my + 1) % N_DEV)
            else:
                cw_ref[s, :, :] = cw_ref[s, :, :] + tA_ref[:, :]
                ccw_ref[s, :, :] = ccw_ref[s, :, :] + tB_ref[:, :]

        ag_slot = [3, 0, 1]
        for s in range(3):
            r_cw = rdma(cw_ref.at[2] if s == 0 else cw_ref.at[ag_slot[s - 1]],
                        cw_ref.at[ag_slot[s]],
                        agcw_send_sems.at[s], agcw_recv_sems.at[s], right)
            r_ccw = rdma(ccw_ref.at[2] if s == 0 else ccw_ref.at[ag_slot[s - 1]],
                         ccw_ref.at[ag_slot[s]],
                         agccw_send_sems.at[s], agccw_recv_sems.at[s], left)
            r_cw.start()
            r_ccw.start()
            if s == 0:
                gemm2_top((my + 1) % N_DEV, cw_ref[2, :, :])
                gemm2_bot((my - 1) % N_DEV, ccw_ref[2, :, :])
            else:
                gemm2_top((my - s + 1) % N_DEV, cw_ref[ag_slot[s - 1], :, :])
                gemm2_bot((my + s - 1) % N_DEV, ccw_ref[ag_slot[s - 1], :, :])
            r_cw.wait()
            r_ccw.wait()
        gemm2_top((my - 2) % N_DEV, cw_ref[ag_slot[2], :, :])
        gemm2_bot((my + 2) % N_DEV, ccw_ref[ag_slot[2], :, :])

    return pl.pallas_call(
        body,
        out_shape=jax.ShapeDtypeStruct((m, n), jnp.float32),
        in_specs=[
            pl.BlockSpec(memory_space=pltpu.VMEM),
            pl.BlockSpec(memory_space=pltpu.VMEM),
            pl.BlockSpec(memory_space=pltpu.VMEM),
        ],
        out_specs=pl.BlockSpec(memory_space=pltpu.VMEM),
        scratch_shapes=[
            pltpu.VMEM((4, HF, d), jnp.float32),
            pltpu.VMEM((4, HF, d), jnp.float32),
            pltpu.VMEM((HF, d), jnp.float32),
            pltpu.VMEM((HF, d), jnp.float32),
            pltpu.SemaphoreType.DMA((3,)),
            pltpu.SemaphoreType.DMA((3,)),
            pltpu.SemaphoreType.DMA((3,)),
            pltpu.SemaphoreType.DMA((3,)),
            pltpu.SemaphoreType.DMA((3,)),
            pltpu.SemaphoreType.DMA((3,)),
            pltpu.SemaphoreType.DMA((3,)),
            pltpu.SemaphoreType.DMA((3,)),
        ],
        compiler_params=pltpu.CompilerParams(
            collective_id=0,
            vmem_limit_bytes=64 * 1024 * 1024,
        ),
    )(x, W1, W2)


# device time: 169929 ns/iter; 1.9436x vs baseline; 1.0772x over previous
import jax
import jax.numpy as jnp
from jax import lax
from jax.experimental import pallas as pl
from jax.experimental.pallas import tpu as pltpu

N_DEV = 4
CH = 1024
HF = CH // 2
SUB = 2
SS = HF // SUB


def kernel(x, W1, W2):
    m, k = x.shape
    d = W1.shape[1]
    n = W2.shape[1]

    def body(x_ref, w1_ref, w2_ref, out_ref, cw_ref, ccw_ref,
             tA_ref, tB_ref,
             cw_send_sems, cw_recv_sems, ccw_send_sems, ccw_recv_sems,
             agcw_send_sems, agcw_recv_sems, agccw_send_sems,
             agccw_recv_sems):
        my = lax.axis_index("i")
        left = (my - 1) % N_DEV
        right = (my + 1) % N_DEV

        barrier_sem = pltpu.get_barrier_semaphore()
        for nbr in [left, right]:
            pl.semaphore_signal(
                barrier_sem, inc=1,
                device_id=(nbr,), device_id_type=pl.DeviceIdType.MESH,
            )
        pl.semaphore_wait(barrier_sem, 2)

        def gemm1_top(c, u):
            return jnp.dot(
                x_ref[pl.ds(c * CH + u * SS, SS), :], w1_ref[:, :],
                preferred_element_type=jnp.float32,
            )

        def gemm1_bot(c, u):
            return jnp.dot(
                x_ref[pl.ds(c * CH + HF + u * SS, SS), :], w1_ref[:, :],
                preferred_element_type=jnp.float32,
            )

        def gemm2_top(c, u, buf):
            out_ref[pl.ds(c * CH + u * SS, SS), :] = jnp.dot(
                buf, w2_ref[:, :], preferred_element_type=jnp.float32,
            )

        def gemm2_bot(c, u, buf):
            out_ref[pl.ds(c * CH + HF + u * SS, SS), :] = jnp.dot(
                buf, w2_ref[:, :], preferred_element_type=jnp.float32,
            )

        def rdma(src, dst, ssem, rsem, target):
            return pltpu.make_async_remote_copy(
                src_ref=src, dst_ref=dst, send_sem=ssem, recv_sem=rsem,
                device_id=(target,), device_id_type=pl.DeviceIdType.MESH,
            )

        def slot(s, u):
            return s * SUB + u

        rs_cw = {}
        rs_ccw = {}

        def make_rs(s, u):
            rs_cw[(s, u)] = rdma(
                tA_ref.at[u] if s == 0 else cw_ref.at[slot(s - 1, u)],
                cw_ref.at[slot(s, u)],
                cw_send_sems.at[slot(s, u)], cw_recv_sems.at[slot(s, u)],
                right)
            rs_ccw[(s, u)] = rdma(
                tB_ref.at[u] if s == 0 else ccw_ref.at[slot(s - 1, u)],
                ccw_ref.at[slot(s, u)],
                ccw_send_sems.at[slot(s, u)], ccw_recv_sems.at[slot(s, u)],
                left)

        for u in range(SUB):
            tA_ref[u, :, :] = gemm1_top(my, u)
            tB_ref[u, :, :] = gemm1_bot(my, u)
            make_rs(0, u)
            rs_cw[(0, u)].start()
            rs_ccw[(0, u)].start()

        ag_base = [3, 0, 1]
        ag_cw = {}
        ag_ccw = {}

        def make_ag(s, u):
            ag_cw[(s, u)] = rdma(
                cw_ref.at[slot(2, u)] if s == 0
                else cw_ref.at[slot(ag_base[s - 1], u)],
                cw_ref.at[slot(ag_base[s], u)],
                agcw_send_sems.at[slot(s, u)], agcw_recv_sems.at[slot(s, u)],
                right)
            ag_ccw[(s, u)] = rdma(
                ccw_ref.at[slot(2, u)] if s == 0
                else ccw_ref.at[slot(ag_base[s - 1], u)],
                ccw_ref.at[slot(ag_base[s], u)],
                agccw_send_sems.at[slot(s, u)],
                agccw_recv_sems.at[slot(s, u)],
                left)

        for s in range(3):
            for u in range(SUB):
                rs_cw[(s, u)].wait()
                cw_ref[slot(s, u), :, :] = (
                    cw_ref[slot(s, u), :, :] + gemm1_top((my - s - 1) % N_DEV, u)
                )
                if s < 2:
                    make_rs(s + 1, u)
                    rs_cw[(s + 1, u)].start()
                else:
                    make_ag(0, u)
                    ag_cw[(0, u)].start()
                rs_ccw[(s, u)].wait()
                ccw_ref[slot(s, u), :, :] = (
                    ccw_ref[slot(s, u), :, :] + gemm1_bot((my + s + 1) % N_DEV, u)
                )
                if s < 2:
                    rs_ccw[(s + 1, u)].start()
                else:
                    ag_ccw[(0, u)].start()

        for u in range(SUB):
            gemm2_top((my + 1) % N_DEV, u, cw_ref[slot(2, u), :, :])
            gemm2_bot((my - 1) % N_DEV, u, ccw_ref[slot(2, u), :, :])

        for s in range(3):
            for u in range(SUB):
                ag_cw[(s, u)].wait()
                if s < 2:
                    make_ag(s + 1, u)
                    ag_cw[(s + 1, u)].start()
                gemm2_top((my - s) % N_DEV, u,
                          cw_ref[slot(ag_base[s], u), :, :])
                ag_ccw[(s, u)].wait()
                if s < 2:
                    ag_ccw[(s + 1, u)].start()
                gemm2_bot((my + s) % N_DEV, u,
                          ccw_ref[slot(ag_base[s], u), :, :])

    return pl.pallas_call(
        body,
        out_shape=jax.ShapeDtypeStruct((m, n), jnp.float32),
        in_specs=[
            pl.BlockSpec(memory_space=pltpu.VMEM),
            pl.BlockSpec(memory_space=pltpu.VMEM),
            pl.BlockSpec(memory_space=pltpu.VMEM),
        ],
        out_specs=pl.BlockSpec(memory_space=pltpu.VMEM),
        scratch_shapes=[
            pltpu.VMEM((4 * SUB, SS, d), jnp.float32),
            pltpu.VMEM((4 * SUB, SS, d), jnp.float32),
            pltpu.VMEM((SUB, SS, d), jnp.float32),
            pltpu.VMEM((SUB, SS, d), jnp.float32),
            pltpu.SemaphoreType.DMA((3 * SUB,)),
            pltpu.SemaphoreType.DMA((3 * SUB,)),
            pltpu.SemaphoreType.DMA((3 * SUB,)),
            pltpu.SemaphoreType.DMA((3 * SUB,)),
            pltpu.SemaphoreType.DMA((3 * SUB,)),
            pltpu.SemaphoreType.DMA((3 * SUB,)),
            pltpu.SemaphoreType.DMA((3 * SUB,)),
            pltpu.SemaphoreType.DMA((3 * SUB,)),
        ],
        compiler_params=pltpu.CompilerParams(
            collective_id=0,
            vmem_limit_bytes=64 * 1024 * 1024,
        ),
    )(x, W1, W2)


# device time: 169216 ns/iter; 1.9518x vs baseline; 1.0042x over previous
import jax
import jax.numpy as jnp
from jax import lax
from jax.experimental import pallas as pl
from jax.experimental.pallas import tpu as pltpu

N_DEV = 4
CH = 1024
HF = CH // 2
SUB = 4
SS = HF // SUB


def kernel(x, W1, W2):
    m, k = x.shape
    d = W1.shape[1]
    n = W2.shape[1]

    def body(x_ref, w1_ref, w2_ref, out_ref, cw_ref, ccw_ref,
             tA_ref, tB_ref,
             cw_send_sems, cw_recv_sems, ccw_send_sems, ccw_recv_sems,
             agcw_send_sems, agcw_recv_sems, agccw_send_sems,
             agccw_recv_sems):
        my = lax.axis_index("i")
        left = (my - 1) % N_DEV
        right = (my + 1) % N_DEV

        barrier_sem = pltpu.get_barrier_semaphore()
        for nbr in [left, right]:
            pl.semaphore_signal(
                barrier_sem, inc=1,
                device_id=(nbr,), device_id_type=pl.DeviceIdType.MESH,
            )
        pl.semaphore_wait(barrier_sem, 2)

        def gemm1_top(c, u):
            return jnp.dot(
                x_ref[pl.ds(c * CH + u * SS, SS), :], w1_ref[:, :],
                preferred_element_type=jnp.float32,
            )

        def gemm1_bot(c, u):
            return jnp.dot(
                x_ref[pl.ds(c * CH + HF + u * SS, SS), :], w1_ref[:, :],
                preferred_element_type=jnp.float32,
            )

        def gemm2_top(c, u, buf):
            out_ref[pl.ds(c * CH + u * SS, SS), :] = jnp.dot(
                buf, w2_ref[:, :], preferred_element_type=jnp.float32,
            )

        def gemm2_bot(c, u, buf):
            out_ref[pl.ds(c * CH + HF + u * SS, SS), :] = jnp.dot(
                buf, w2_ref[:, :], preferred_element_type=jnp.float32,
            )

        def rdma(src, dst, ssem, rsem, target):
            return pltpu.make_async_remote_copy(
                src_ref=src, dst_ref=dst, send_sem=ssem, recv_sem=rsem,
                device_id=(target,), device_id_type=pl.DeviceIdType.MESH,
            )

        def slot(s, u):
            return s * SUB + u

        rs_cw = {}
        rs_ccw = {}

        def make_rs(s, u):
            rs_cw[(s, u)] = rdma(
                tA_ref.at[u] if s == 0 else cw_ref.at[slot(s - 1, u)],
                cw_ref.at[slot(s, u)],
                cw_send_sems.at[slot(s, u)], cw_recv_sems.at[slot(s, u)],
                right)
            rs_ccw[(s, u)] = rdma(
                tB_ref.at[u] if s == 0 else ccw_ref.at[slot(s - 1, u)],
                ccw_ref.at[slot(s, u)],
                ccw_send_sems.at[slot(s, u)], ccw_recv_sems.at[slot(s, u)],
                left)

        for u in range(SUB):
            tA_ref[u, :, :] = gemm1_top(my, u)
            tB_ref[u, :, :] = gemm1_bot(my, u)
            make_rs(0, u)
            rs_cw[(0, u)].start()
            rs_ccw[(0, u)].start()

        ag_base = [3, 0, 1]
        ag_cw = {}
        ag_ccw = {}

        def make_ag(s, u):
            ag_cw[(s, u)] = rdma(
                cw_ref.at[slot(2, u)] if s == 0
                else cw_ref.at[slot(ag_base[s - 1], u)],
                cw_ref.at[slot(ag_base[s], u)],
                agcw_send_sems.at[slot(s, u)], agcw_recv_sems.at[slot(s, u)],
                right)
            ag_ccw[(s, u)] = rdma(
                ccw_ref.at[slot(2, u)] if s == 0
                else ccw_ref.at[slot(ag_base[s - 1], u)],
                ccw_ref.at[slot(ag_base[s], u)],
                agccw_send_sems.at[slot(s, u)],
                agccw_recv_sems.at[slot(s, u)],
                left)

        for s in range(3):
            for u in range(SUB):
                rs_cw[(s, u)].wait()
                cw_ref[slot(s, u), :, :] = (
                    cw_ref[slot(s, u), :, :] + gemm1_top((my - s - 1) % N_DEV, u)
                )
                if s < 2:
                    make_rs(s + 1, u)
                    rs_cw[(s + 1, u)].start()
                else:
                    make_ag(0, u)
                    ag_cw[(0, u)].start()
                rs_ccw[(s, u)].wait()
                ccw_ref[slot(s, u), :, :] = (
                    ccw_ref[slot(s, u), :, :] + gemm1_bot((my + s + 1) % N_DEV, u)
                )
                if s < 2:
                    rs_ccw[(s + 1, u)].start()
                else:
                    ag_ccw[(0, u)].start()

        for u in range(SUB):
            gemm2_top((my + 1) % N_DEV, u, cw_ref[slot(2, u), :, :])
            gemm2_bot((my - 1) % N_DEV, u, ccw_ref[slot(2, u), :, :])

        for s in range(3):
            for u in range(SUB):
                ag_cw[(s, u)].wait()
                if s < 2:
                    make_ag(s + 1, u)
                    ag_cw[(s + 1, u)].start()
                gemm2_top((my - s) % N_DEV, u,
                          cw_ref[slot(ag_base[s], u), :, :])
                ag_ccw[(s, u)].wait()
                if s < 2:
                    ag_ccw[(s + 1, u)].start()
                gemm2_bot((my + s) % N_DEV, u,
                          ccw_ref[slot(ag_base[s], u), :, :])

    return pl.pallas_call(
        body,
        out_shape=jax.ShapeDtypeStruct((m, n), jnp.float32),
        in_specs=[
            pl.BlockSpec(memory_space=pltpu.VMEM),
            pl.BlockSpec(memory_space=pltpu.VMEM),
            pl.BlockSpec(memory_space=pltpu.VMEM),
        ],
        out_specs=pl.BlockSpec(memory_space=pltpu.VMEM),
        scratch_shapes=[
            pltpu.VMEM((4 * SUB, SS, d), jnp.float32),
            pltpu.VMEM((4 * SUB, SS, d), jnp.float32),
            pltpu.VMEM((SUB, SS, d), jnp.float32),
            pltpu.VMEM((SUB, SS, d), jnp.float32),
            pltpu.SemaphoreType.DMA((3 * SUB,)),
            pltpu.SemaphoreType.DMA((3 * SUB,)),
            pltpu.SemaphoreType.DMA((3 * SUB,)),
            pltpu.SemaphoreType.DMA((3 * SUB,)),
            pltpu.SemaphoreType.DMA((3 * SUB,)),
            pltpu.SemaphoreType.DMA((3 * SUB,)),
            pltpu.SemaphoreType.DMA((3 * SUB,)),
            pltpu.SemaphoreType.DMA((3 * SUB,)),
        ],
        compiler_params=pltpu.CompilerParams(
            collective_id=0,
            vmem_limit_bytes=64 * 1024 * 1024,
        ),
    )(x, W1, W2)
